# baseline (speedup 1.0000x reference)
import functools
import numpy as np
import jax
import jax.numpy as jnp
from jax import lax
from jax.experimental import pallas as pl
from jax.experimental.pallas import tpu as pltpu
from jax.experimental.pallas import tpu_sc as plsc

_RATIOS = np.array([0.5, 1.0, 2.0], dtype=np.float32)
_SCALES = np.array([1.0, 2.0 ** (1.0 / 3.0), 2.0 ** (2.0 / 3.0)], dtype=np.float32)
_SCALES_REP = np.tile(_SCALES, 3)
_RATIOS_REP = np.repeat(_RATIOS, 3)
_W0 = ((np.float32(32.0) * _SCALES_REP) / np.sqrt(_RATIOS_REP)).astype(np.float32)
_H0 = (_W0 * _RATIOS_REP).astype(np.float32)

_N = 48960
_OFF1, _OFF2, _OFF3 = 36864, 46080, 48384
_CHUNK = 1536
_TAIL_BASE = 31 * _CHUNK
_TAIL = 1280
_END_BASE = _TAIL_BASE + _TAIL


def _values4(n):
    c1, c2, c3 = n >= _OFF1, n >= _OFF2, n >= _OFF3
    offset = jnp.where(c3, _OFF3, jnp.where(c2, _OFF2, jnp.where(c1, _OFF1, 0)))
    local = n - offset
    q = (local * 58255) >> 19
    a = local - q * 9
    hh = jnp.where(c3, q >> 3, jnp.where(c2, q >> 4, jnp.where(c1, q >> 5, q >> 6)))
    mask = jnp.where(c3, 7, jnp.where(c2, 15, jnp.where(c1, 31, 63)))
    ww = q & mask
    stride = jnp.where(c3, 64.0, jnp.where(c2, 32.0, jnp.where(c1, 16.0, 8.0)))
    s2l = jnp.where(c3, 8.0, jnp.where(c2, 4.0, jnp.where(c1, 2.0, 1.0)))
    x = (ww.astype(jnp.float32) + 0.5) * stride
    y = (hh.astype(jnp.float32) + 0.5) * stride
    wa = jnp.full_like(x, float(_W0[8]))
    ha = jnp.full_like(x, float(_H0[8]))
    for i in range(7, -1, -1):
        wa = jnp.where(a == i, float(_W0[i]), wa)
        ha = jnp.where(a == i, float(_H0[i]), ha)
    return x, y, wa * s2l, ha * s2l


def _fill(buf, base, count):
    def step(v, _):
        n = base + v * 16 + lax.iota(jnp.int32, 16)
        x, y, wa, ha = _values4(n)
        sl = pl.ds(v * 16, 16)
        buf[0, sl] = x
        buf[1, sl] = y
        buf[2, sl] = wa
        buf[3, sl] = ha
        return 0

    lax.fori_loop(0, count // 16, step, 0)


def _sc_body(out_hbm, buf, end):
    wid = lax.axis_index("s") * 2 + lax.axis_index("c")

    @pl.when(wid < 31)
    def _():
        base = pl.multiple_of(wid * _CHUNK, 128)
        _fill(buf, base, _CHUNK)
        pltpu.sync_copy(buf.at[:, pl.ds(0, _CHUNK)],
                        out_hbm.at[:, pl.ds(base, _CHUNK)])

    @pl.when(wid == 31)
    def _():
        _fill(buf, _TAIL_BASE, _TAIL)
        pltpu.sync_copy(buf.at[:, pl.ds(0, _TAIL)],
                        out_hbm.at[:, pl.ds(_TAIL_BASE, _TAIL)])
        _fill(end, _END_BASE, 64)
        pltpu.sync_copy(end, out_hbm.at[:, pl.ds(_END_BASE, 64)])


def sc_anchors():
    mesh = plsc.VectorSubcoreMesh(core_axis_name="c", subcore_axis_name="s")
    k = functools.partial(
        pl.kernel,
        out_type=jax.ShapeDtypeStruct((4, _N), jnp.float32),
        mesh=mesh,
        scratch_types=[pltpu.VMEM((4, _CHUNK), jnp.float32),
                       pltpu.VMEM((4, 64), jnp.float32)],
    )(_sc_body)
    return k()


def kernel(feat0, feat1, feat2, feat3):
    del feat0, feat1, feat2, feat3
    return sc_anchors().T


# speedup vs baseline: 1.5216x; 1.5216x over previous
import numpy as np
import jax
import jax.numpy as jnp
from jax import lax
from jax.experimental import pallas as pl
from jax.experimental.pallas import tpu as pltpu

_RATIOS = np.array([0.5, 1.0, 2.0], dtype=np.float32)
_SCALES = np.array([1.0, 2.0 ** (1.0 / 3.0), 2.0 ** (2.0 / 3.0)], dtype=np.float32)
_SCALES_REP = np.tile(_SCALES, 3)
_RATIOS_REP = np.repeat(_RATIOS, 3)
_W0 = ((np.float32(32.0) * _SCALES_REP) / np.sqrt(_RATIOS_REP)).astype(np.float32)
_H0 = (_W0 * _RATIOS_REP).astype(np.float32)

_N = 48960
_OFF1, _OFF2, _OFF3 = 36864, 46080, 48384
_ST = 1152
_NB = 43


def _values(n, c):
    lvl = ((n >= _OFF1).astype(jnp.int32)
           + (n >= _OFF2).astype(jnp.int32)
           + (n >= _OFF3).astype(jnp.int32))
    offset = jnp.where(lvl == 0, 0,
              jnp.where(lvl == 1, _OFF1,
               jnp.where(lvl == 2, _OFF2, _OFF3)))
    local = n - offset
    q = local // 9
    a = local - q * 9
    log2w = 6 - lvl
    hh = q >> log2w
    ww = q & ((1 << log2w) - 1)
    s2l = jnp.where(lvl == 0, 1.0,
           jnp.where(lvl == 1, 2.0,
            jnp.where(lvl == 2, 4.0, 8.0)))
    stride = 8.0 * s2l
    x = (ww.astype(jnp.float32) + 0.5) * stride
    y = (hh.astype(jnp.float32) + 0.5) * stride
    wa = jnp.full_like(x, float(_W0[8]))
    ha = jnp.full_like(x, float(_H0[8]))
    for i in range(7, -1, -1):
        wa = jnp.where(a == i, float(_W0[i]), wa)
        ha = jnp.where(a == i, float(_H0[i]), ha)
    wa = wa * s2l
    ha = ha * s2l
    return jnp.where(c == 0, x,
            jnp.where(c == 1, y,
             jnp.where(c == 2, wa, ha)))


def _block_values(anchor_base):
    n = anchor_base + lax.broadcasted_iota(jnp.int32, (4, _ST), 1)
    c = lax.broadcasted_iota(jnp.int32, (4, _ST), 0)
    return _values(n, c)


def _body(out_ref, pat_ref):
    i = pl.program_id(0)

    @pl.when(i == 0)
    def _():
        pat_ref[0] = _block_values(0)
        pat_ref[1] = _block_values(_OFF1)
        pat_ref[2] = _block_values(_OFF2)

    @pl.when(i < _NB - 1)
    def _():
        lvl = ((i >= 32).astype(jnp.int32) + (i >= 40).astype(jnp.int32))
        tloc = i - jnp.where(lvl == 1, 32, jnp.where(lvl == 2, 40, 0))
        coef = jnp.where(lvl == 1, 64.0, jnp.where(lvl == 2, 256.0, 16.0))
        ybase = coef * tloc.astype(jnp.float32)
        ymask = (lax.broadcasted_iota(jnp.int32, (4, _ST), 0) == 1
                 ).astype(jnp.float32)
        out_ref[...] = pat_ref[lvl] + ybase * ymask

    @pl.when(i == _NB - 1)
    def _():
        out_ref[...] = _block_values(_OFF3)


def kernel(feat0, feat1, feat2, feat3):
    del feat0, feat1, feat2, feat3
    t = pl.pallas_call(
        _body,
        grid=(_NB,),
        out_specs=pl.BlockSpec((4, _ST), lambda i: (0, i)),
        out_shape=jax.ShapeDtypeStruct((4, _N), jnp.float32),
        scratch_shapes=[pltpu.VMEM((3, 4, _ST), jnp.float32)],
    )()
    return t.T


# speedup vs baseline: 17.7286x; 11.6511x over previous
import numpy as np
import jax
import jax.numpy as jnp
from jax import lax
from jax.experimental import pallas as pl

_RATIOS = np.array([0.5, 1.0, 2.0], dtype=np.float32)
_SCALES = np.array([1.0, 2.0 ** (1.0 / 3.0), 2.0 ** (2.0 / 3.0)], dtype=np.float32)
_SCALES_REP = np.tile(_SCALES, 3)
_RATIOS_REP = np.repeat(_RATIOS, 3)
_W0 = ((np.float32(32.0) * _SCALES_REP) / np.sqrt(_RATIOS_REP)).astype(np.float32)
_H0 = (_W0 * _RATIOS_REP).astype(np.float32)

_N = 48960
_OFF1, _OFF2, _OFF3 = 36864, 46080, 48384
_ST = 1152


def _values(n, c):
    lvl = ((n >= _OFF1).astype(jnp.int32)
           + (n >= _OFF2).astype(jnp.int32)
           + (n >= _OFF3).astype(jnp.int32))
    offset = jnp.where(lvl == 0, 0,
              jnp.where(lvl == 1, _OFF1,
               jnp.where(lvl == 2, _OFF2, _OFF3)))
    local = n - offset
    q = local // 9
    a = local - q * 9
    log2w = 6 - lvl
    hh = q >> log2w
    ww = q & ((1 << log2w) - 1)
    s2l = jnp.where(lvl == 0, 1.0,
           jnp.where(lvl == 1, 2.0,
            jnp.where(lvl == 2, 4.0, 8.0)))
    stride = 8.0 * s2l
    x = (ww.astype(jnp.float32) + 0.5) * stride
    y = (hh.astype(jnp.float32) + 0.5) * stride
    wa = jnp.full_like(x, float(_W0[8]))
    ha = jnp.full_like(x, float(_H0[8]))
    for i in range(7, -1, -1):
        wa = jnp.where(a == i, float(_W0[i]), wa)
        ha = jnp.where(a == i, float(_H0[i]), ha)
    wa = wa * s2l
    ha = ha * s2l
    return jnp.where(c == 0, x,
            jnp.where(c == 1, y,
             jnp.where(c == 2, wa, ha)))


def _pattern(base):
    n = base + lax.broadcasted_iota(jnp.int32, (4, _ST), 1)
    c = lax.broadcasted_iota(jnp.int32, (4, _ST), 0)
    return _values(n, c)


def _body(out_ref):
    ymask = (lax.broadcasted_iota(jnp.int32, (4, _ST), 0) == 1).astype(jnp.float32)
    for base, cnt, coef in ((0, 32, 16.0), (_OFF1, 8, 64.0), (_OFF2, 2, 256.0)):
        pat = _pattern(base)
        for t in range(cnt):
            off = base + t * _ST
            out_ref[:, off:off + _ST] = pat + (coef * t) * ymask
    out_ref[:, _OFF3:_N] = _pattern(_OFF3)[:, : _N - _OFF3]


def kernel(feat0, feat1, feat2, feat3):
    del feat0, feat1, feat2, feat3
    t = pl.pallas_call(
        _body,
        out_shape=jax.ShapeDtypeStruct((4, _N), jnp.float32),
    )()
    return t.T
